# Initial kernel scaffold; baseline (speedup 1.0000x reference)
import jax
import jax.numpy as jnp
from jax.experimental import pallas as pl


def kernel(x, edge_index, edge_attr, batch, W1_rel, b1_rel, W1_root, W2_rel, b2_rel, W2_root, W_lin1, b_lin1, W_lin2, b_lin2):
    raise NotImplementedError("write your pallas kernel here")


# speedup vs baseline: 5.5413x; 5.5413x over previous
import functools

import jax
import jax.numpy as jnp
from jax import lax
from jax.experimental import pallas as pl
from jax.experimental.pallas import tpu as pltpu
from jax.experimental.pallas import tpu_sc as plsc

N = 10000
E = 320000
D = 128
G = 64

NC = 2
NS = 16
NW = NC * NS
C = 125
EPW = E // NW
KCH = EPW // C
RPT = N // NS
ZR = 25
WPT = 632
WLAST = N - (NS - 1) * WPT

BLK = 400
NBLK = N // BLK


def _spmm_body(x_hbm, src_hbm, dst_hbm, w_hbm, out_hbm,
               acc, sidx, didx, wbuf, rows, zbuf, gsem):
    cid = lax.axis_index("c")
    sid = lax.axis_index("s")
    wid = cid * NS + sid

    zero16 = jnp.zeros((16,), jnp.float32)
    for r in range(ZR):
        for j in range(8):
            zbuf[r, pl.ds(16 * j, 16)] = zero16
    for k in range(RPT // ZR):
        pltpu.sync_copy(zbuf, acc.at[pl.ds(sid * RPT + k * ZR, ZR)])

    pltpu.sync_copy(src_hbm.at[pl.ds(wid * KCH, KCH)], sidx)
    pltpu.sync_copy(dst_hbm.at[pl.ds(wid * KCH, KCH)], didx)
    pltpu.sync_copy(w_hbm.at[pl.ds(wid * KCH, KCH)], wbuf)
    plsc.subcore_barrier()

    def chunk(k, carry):
        pltpu.async_copy(x_hbm.at[sidx.at[k]], rows, gsem).wait()

        def scale(i, c2):
            ws = plsc.load_gather(wbuf, [jnp.full((16,), k, jnp.int32),
                                         jnp.full((16,), i, jnp.int32)])
            for j in range(8):
                sl = pl.ds(16 * j, 16)
                rows[i, sl] = rows[i, sl] * ws
            return c2
        lax.fori_loop(0, C, scale, 0, unroll=2)

        pltpu.sync_copy(rows, acc.at[didx.at[k]], add=True)
        return carry
    lax.fori_loop(0, KCH, chunk, 0)

    plsc.subcore_barrier()

    @pl.when(sid < NS - 1)
    def _wmain():
        pltpu.sync_copy(acc.at[pl.ds(sid * WPT, WPT)],
                        out_hbm.at[cid, pl.ds(sid * WPT, WPT)])

    @pl.when(sid == NS - 1)
    def _wtail():
        pltpu.sync_copy(acc.at[pl.ds((NS - 1) * WPT, WLAST)],
                        out_hbm.at[cid, pl.ds((NS - 1) * WPT, WLAST)])


def _spmm_sc(x, src2, dst2, w2):
    mesh = plsc.VectorSubcoreMesh(core_axis_name="c", subcore_axis_name="s",
                                  num_cores=NC, num_subcores=NS)
    f = pl.kernel(
        _spmm_body,
        out_type=jax.ShapeDtypeStruct((NC, N, D), jnp.float32),
        mesh=mesh,
        compiler_params=pltpu.CompilerParams(use_tc_tiling_on_sc=False,
                                             needs_layout_passes=False),
        scratch_types=[
            pltpu.VMEM_SHARED((N, D), jnp.float32),
            pltpu.VMEM((KCH, C), jnp.int32),
            pltpu.VMEM((KCH, C), jnp.int32),
            pltpu.VMEM((KCH, C), jnp.float32),
            pltpu.VMEM((C, D), jnp.float32),
            pltpu.VMEM((ZR, D), jnp.float32),
            pltpu.SemaphoreType.DMA,
        ],
    )
    return f(x, src2, dst2, w2)


def _dotT(a, b):
    return lax.dot_general(a, b, (((1,), (1,)), ((), ())),
                           preferred_element_type=jnp.float32)


def _dense_body(a0, a1, xb, wr, wt, br, out):
    s = a0[...] + a1[...]
    h = _dotT(s, wr[...]) + _dotT(xb[...], wt[...]) + br[...]
    out[...] = jnp.maximum(h, 0.0)


def _dense_tc(agg, x, W_rel, b_rel, W_root):
    b2d = b_rel.reshape(1, D)
    grid = (NBLK,)
    return pl.pallas_call(
        _dense_body,
        grid=grid,
        in_specs=[
            pl.BlockSpec((BLK, D), lambda k: (k, 0)),
            pl.BlockSpec((BLK, D), lambda k: (k, 0)),
            pl.BlockSpec((BLK, D), lambda k: (k, 0)),
            pl.BlockSpec((D, D), lambda k: (0, 0)),
            pl.BlockSpec((D, D), lambda k: (0, 0)),
            pl.BlockSpec((1, D), lambda k: (0, 0)),
        ],
        out_specs=pl.BlockSpec((BLK, D), lambda k: (k, 0)),
        out_shape=jax.ShapeDtypeStruct((N, D), jnp.float32),
    )(agg[0], agg[1], x, W_rel, W_root, b2d)


def _head_body(a0, a1, hb, wr, wt, br, batchb, wl1, bl1, wl2, bl2,
               out, psum, cnt):
    k = pl.program_id(0)

    @pl.when(k == 0)
    def _init():
        psum[...] = jnp.zeros_like(psum)
        cnt[...] = jnp.zeros_like(cnt)

    s = a0[...] + a1[...]
    h = _dotT(s, wr[...]) + _dotT(hb[...], wt[...]) + br[...]
    h = jnp.maximum(h, 0.0)
    bvec = batchb[0, 0, :]
    onehot = (lax.broadcasted_iota(jnp.int32, (G, BLK), 0)
              == bvec[None, :]).astype(jnp.float32)
    psum[...] += jnp.dot(onehot, h, preferred_element_type=jnp.float32)
    cnt[...] += jnp.broadcast_to(
        jnp.sum(onehot, axis=1, keepdims=True), (G, D))

    @pl.when(k == NBLK - 1)
    def _fin():
        p = psum[...] / jnp.maximum(cnt[...], 1.0)
        z = jnp.maximum(_dotT(p, wl1[...]) + bl1[...], 0.0)
        out[...] = _dotT(z, wl2[...]) + bl2[...]


def _head_tc(agg, h1, W_rel, b_rel, W_root, batchr, W_lin1, b_lin1,
             W_lin2p, b_lin2p):
    b2d = b_rel.reshape(1, D)
    grid = (NBLK,)
    return pl.pallas_call(
        _head_body,
        grid=grid,
        in_specs=[
            pl.BlockSpec((BLK, D), lambda k: (k, 0)),
            pl.BlockSpec((BLK, D), lambda k: (k, 0)),
            pl.BlockSpec((BLK, D), lambda k: (k, 0)),
            pl.BlockSpec((D, D), lambda k: (0, 0)),
            pl.BlockSpec((D, D), lambda k: (0, 0)),
            pl.BlockSpec((1, D), lambda k: (0, 0)),
            pl.BlockSpec((1, 1, BLK), lambda k: (k, 0, 0)),
            pl.BlockSpec((16, D), lambda k: (0, 0)),
            pl.BlockSpec((1, 16), lambda k: (0, 0)),
            pl.BlockSpec((16, 16), lambda k: (0, 0)),
            pl.BlockSpec((1, 16), lambda k: (0, 0)),
        ],
        out_specs=pl.BlockSpec((G, 16), lambda k: (0, 0)),
        out_shape=jax.ShapeDtypeStruct((G, 16), jnp.float32),
        scratch_shapes=[
            pltpu.VMEM((G, D), jnp.float32),
            pltpu.VMEM((G, D), jnp.float32),
        ],
    )(agg[0], agg[1], h1, W_rel, W_root, b2d, batchr,
      W_lin1, b_lin1.reshape(1, 16), W_lin2p, b_lin2p)


def kernel(x, edge_index, edge_attr, batch, W1_rel, b1_rel, W1_root,
           W2_rel, b2_rel, W2_root, W_lin1, b_lin1, W_lin2, b_lin2):
    src2 = edge_index[0].reshape(E // C, C)
    dst2 = edge_index[1].reshape(E // C, C)
    w2 = edge_attr.reshape(E // C, C)
    batchr = batch.reshape(NBLK, 1, BLK)
    W_lin2p = jnp.zeros((16, 16), jnp.float32).at[0].set(W_lin2[0])
    b_lin2p = jnp.zeros((1, 16), jnp.float32).at[0, 0].set(b_lin2[0])

    agg1 = _spmm_sc(x, src2, dst2, w2)
    h1 = _dense_tc(agg1, x, W1_rel, b1_rel, W1_root)
    agg2 = _spmm_sc(h1, src2, dst2, w2)
    out16 = _head_tc(agg2, h1, W2_rel, b2_rel, W2_root, batchr,
                     W_lin1, b_lin1, W_lin2p, b_lin2p)
    return out16[:, 0:1]
